# Initial kernel scaffold; baseline (speedup 1.0000x reference)
import jax
import jax.numpy as jnp
from jax.experimental import pallas as pl


def kernel(input_ids, sentences_ids, W_embed, nomal_emb, W1, b1, W2, b2):
    raise NotImplementedError("write your pallas kernel here")


# speedup vs baseline: 1.5166x; 1.5166x over previous
import functools

import jax
import jax.numpy as jnp
from jax import lax
from jax.experimental import pallas as pl
from jax.experimental.pallas import tpu as pltpu
from jax.experimental.pallas import tpu_sc as plsc

_VOCAB = 100000
_HIDDEN = 768
_PRE_SEQ_LEN = 128
_BATCH = 16
_SEQ = 2048
_PAD_ID = 0
_UNK_ID = 1
_SPECIAL_ID = 99999

_NC = 2
_NS = 16
_L = 16
_NW = _NC * _NS
_B = _BATCH * _SEQ
_BPW = _B // _NW
_C = 64
_NCHUNK = _BPW // _C

_sc_mesh = plsc.VectorSubcoreMesh(
    core_axis_name="c", subcore_axis_name="s",
    num_cores=_NC, num_subcores=_NS,
)


@functools.partial(
    pl.kernel,
    out_type=jax.ShapeDtypeStruct((_B, _HIDDEN), jnp.float32),
    mesh=_sc_mesh,
    scratch_types=[
        pltpu.VMEM((_NCHUNK, _C), jnp.int32),
        pltpu.VMEM((2, _C, _HIDDEN), jnp.float32),
        pltpu.SemaphoreType.DMA,
    ],
)
def _gather_sc(ids_hbm, table_hbm, out_hbm, idx_v, rows_v, gsem):
    wid = lax.axis_index("s") * _NC + lax.axis_index("c")
    base = wid * _BPW
    pltpu.sync_copy(ids_hbm.at[wid], idx_v)
    for t in range(_NCHUNK):
        row = idx_v.at[t]
        for j in range(_C // _L):
            v = row[pl.ds(j * _L, _L)]
            row[pl.ds(j * _L, _L)] = jnp.where(v == _SPECIAL_ID, _UNK_ID, v)
    for t in range(_NCHUNK):
        buf = t % 2
        pltpu.async_copy(table_hbm.at[idx_v.at[t]], rows_v.at[buf], gsem).wait()
        pltpu.sync_copy(rows_v.at[buf], out_hbm.at[pl.ds(base + t * _C, _C)])


def _mlp_mask_body(ids_ref, emb_ref, w1_ref, b1_ref, w2_ref, b2_ref,
                   prompts_ref, mask_ref):
    h = jnp.dot(emb_ref[...], w1_ref[...], preferred_element_type=jnp.float32)
    h = jnp.maximum(h + b1_ref[...], 0.0)
    h = jnp.dot(h, w2_ref[...], preferred_element_type=jnp.float32)
    h = jnp.maximum(h + b2_ref[...], 0.0)
    prompts_ref[...] = jnp.broadcast_to(h[None], (_BATCH, _PRE_SEQ_LEN, _HIDDEN))
    mask_ref[...] = (ids_ref[...] != _PAD_ID).astype(jnp.int8)


_mlp_mask = pl.pallas_call(
    _mlp_mask_body,
    out_shape=(
        jax.ShapeDtypeStruct((_BATCH, _PRE_SEQ_LEN, _HIDDEN), jnp.float32),
        jax.ShapeDtypeStruct((_BATCH, _SEQ), jnp.int8),
    ),
)


def kernel(input_ids, sentences_ids, W_embed, nomal_emb, W1, b1, W2, b2):
    ids32 = input_ids.astype(jnp.int32)
    ids_grp = ids32.reshape(_NW, _NCHUNK, _C)
    flat = _gather_sc(ids_grp, W_embed)
    inputs_embeds = flat.reshape(_BATCH, _SEQ, _HIDDEN)
    prompts, mask8 = _mlp_mask(
        ids32, nomal_emb, W1, b1.reshape(1, _HIDDEN), W2, b2.reshape(1, _HIDDEN)
    )
    return inputs_embeds, prompts, mask8.astype(jnp.bool_)


# speedup vs baseline: 1.6573x; 1.0928x over previous
import functools

import jax
import jax.numpy as jnp
from jax import lax
from jax.experimental import pallas as pl
from jax.experimental.pallas import tpu as pltpu
from jax.experimental.pallas import tpu_sc as plsc

_VOCAB = 100000
_HIDDEN = 768
_PRE_SEQ_LEN = 128
_BATCH = 16
_SEQ = 2048
_PAD_ID = 0
_UNK_ID = 1
_SPECIAL_ID = 99999

_NC = 2
_NS = 16
_L = 16
_NW = _NC * _NS
_B = _BATCH * _SEQ
_BPW = _B // _NW
_C = 64
_NCHUNK = _BPW // _C

_sc_mesh = plsc.VectorSubcoreMesh(
    core_axis_name="c", subcore_axis_name="s",
    num_cores=_NC, num_subcores=_NS,
)


@functools.partial(
    pl.kernel,
    out_type=jax.ShapeDtypeStruct((_B, _HIDDEN), jnp.float32),
    mesh=_sc_mesh,
    scratch_types=[
        pltpu.VMEM((_NCHUNK, _C), jnp.int32),
        pltpu.VMEM((2, _C, _HIDDEN), jnp.float32),
        pltpu.SemaphoreType.DMA,
        pltpu.SemaphoreType.DMA,
        pltpu.SemaphoreType.DMA,
        pltpu.SemaphoreType.DMA,
    ],
)
def _gather_sc(ids_hbm, table_hbm, out_hbm, idx_v, rows_v,
               gsem0, gsem1, osem0, osem1):
    wid = lax.axis_index("s") * _NC + lax.axis_index("c")
    base = wid * _BPW
    gsem = (gsem0, gsem1)
    osem = (osem0, osem1)
    pltpu.sync_copy(ids_hbm.at[wid], idx_v)
    for t in range(_NCHUNK):
        row = idx_v.at[t]
        for j in range(_C // _L):
            v = row[pl.ds(j * _L, _L)]
            row[pl.ds(j * _L, _L)] = jnp.where(v == _SPECIAL_ID, _UNK_ID, v)

    def start_gather(t):
        buf = t % 2
        return pltpu.async_copy(
            table_hbm.at[idx_v.at[t]], rows_v.at[buf], gsem[buf])

    def start_out(t):
        buf = t % 2
        return pltpu.async_copy(
            rows_v.at[buf], out_hbm.at[pl.ds(base + t * _C, _C)], osem[buf])

    out_pending = [None, None]
    g_cur = start_gather(0)
    for t in range(_NCHUNK):
        buf = t % 2
        g_cur.wait()
        if t + 1 < _NCHUNK:
            nbuf = (t + 1) % 2
            if out_pending[nbuf] is not None:
                out_pending[nbuf].wait()
                out_pending[nbuf] = None
            g_cur = start_gather(t + 1)
        out_pending[buf] = start_out(t)
    for h in out_pending:
        if h is not None:
            h.wait()


def _mlp_mask_body(ids_ref, emb_ref, w1_ref, b1_ref, w2_ref, b2_ref,
                   prompts_ref, mask_ref):
    h = jnp.dot(emb_ref[...], w1_ref[...], preferred_element_type=jnp.float32)
    h = jnp.maximum(h + b1_ref[...], 0.0)
    h = jnp.dot(h, w2_ref[...], preferred_element_type=jnp.float32)
    h = jnp.maximum(h + b2_ref[...], 0.0)
    prompts_ref[...] = jnp.broadcast_to(h[None], (_BATCH, _PRE_SEQ_LEN, _HIDDEN))
    mask_ref[...] = (ids_ref[...] != _PAD_ID).astype(jnp.int8)


_mlp_mask = pl.pallas_call(
    _mlp_mask_body,
    out_shape=(
        jax.ShapeDtypeStruct((_BATCH, _PRE_SEQ_LEN, _HIDDEN), jnp.float32),
        jax.ShapeDtypeStruct((_BATCH, _SEQ), jnp.int8),
    ),
)


def kernel(input_ids, sentences_ids, W_embed, nomal_emb, W1, b1, W2, b2):
    ids32 = input_ids.astype(jnp.int32)
    ids_grp = ids32.reshape(_NW, _NCHUNK, _C)
    flat = _gather_sc(ids_grp, W_embed)
    inputs_embeds = flat.reshape(_BATCH, _SEQ, _HIDDEN)
    prompts, mask8 = _mlp_mask(
        ids32, nomal_emb, W1, b1.reshape(1, _HIDDEN), W2, b2.reshape(1, _HIDDEN)
    )
    return inputs_embeds, prompts, mask8.astype(jnp.bool_)


# speedup vs baseline: 1.6765x; 1.0116x over previous
import functools

import jax
import jax.numpy as jnp
from jax import lax
from jax.experimental import pallas as pl
from jax.experimental.pallas import tpu as pltpu
from jax.experimental.pallas import tpu_sc as plsc

_VOCAB = 100000
_HIDDEN = 768
_PRE_SEQ_LEN = 128
_BATCH = 16
_SEQ = 2048
_PAD_ID = 0
_UNK_ID = 1
_SPECIAL_ID = 99999

_NC = 2
_NS = 16
_L = 16
_NW = _NC * _NS
_B = _BATCH * _SEQ
_BPW = _B // _NW
_C = 32
_NCHUNK = _BPW // _C
_NBUF = 4

_sc_mesh = plsc.VectorSubcoreMesh(
    core_axis_name="c", subcore_axis_name="s",
    num_cores=_NC, num_subcores=_NS,
)


@functools.partial(
    pl.kernel,
    out_type=jax.ShapeDtypeStruct((_B, _HIDDEN), jnp.float32),
    mesh=_sc_mesh,
    scratch_types=[
        pltpu.VMEM((_BPW,), jnp.int32),
        pltpu.VMEM((_NBUF, _C, _HIDDEN), jnp.float32),
        [pltpu.SemaphoreType.DMA] * _NBUF,
        [pltpu.SemaphoreType.DMA] * _NBUF,
    ],
)
def _gather_sc(ids_hbm, table_hbm, out_hbm, idx_v, rows_v, gsem, osem):
    wid = lax.axis_index("s") * _NC + lax.axis_index("c")
    b = wid // 2
    half = wid % 2
    base = wid * _BPW
    pltpu.sync_copy(ids_hbm.at[b, pl.ds(half * _BPW, _BPW)], idx_v)
    for j in range(_BPW // _L):
        v = idx_v[pl.ds(j * _L, _L)]
        idx_v[pl.ds(j * _L, _L)] = jnp.where(v == _SPECIAL_ID, _UNK_ID, v)

    def start_gather(t):
        buf = t % _NBUF
        return pltpu.async_copy(
            table_hbm.at[idx_v.at[pl.ds(t * _C, _C)]], rows_v.at[buf],
            gsem[buf])

    def start_out(t):
        buf = t % _NBUF
        return pltpu.async_copy(
            rows_v.at[buf], out_hbm.at[pl.ds(base + t * _C, _C)], osem[buf])

    out_pending = [None] * _NBUF
    g_pending = [None] * _NCHUNK
    g_pending[0] = start_gather(0)
    g_pending[1] = start_gather(1)
    for t in range(_NCHUNK):
        g_pending[t].wait()
        nt = t + 2
        if nt < _NCHUNK:
            nbuf = nt % _NBUF
            if out_pending[nbuf] is not None:
                out_pending[nbuf].wait()
                out_pending[nbuf] = None
            g_pending[nt] = start_gather(nt)
        out_pending[t % _NBUF] = start_out(t)
    for h in out_pending:
        if h is not None:
            h.wait()


def _mlp_mask_body(ids_ref, emb_ref, w1_ref, b1_ref, w2_ref, b2_ref,
                   prompts_ref, mask_ref):
    h = jnp.dot(emb_ref[...], w1_ref[...], preferred_element_type=jnp.float32)
    h = jnp.maximum(h + b1_ref[...], 0.0)
    h = jnp.dot(h, w2_ref[...], preferred_element_type=jnp.float32)
    h = jnp.maximum(h + b2_ref[...], 0.0)
    prompts_ref[...] = jnp.broadcast_to(h[None], (_BATCH, _PRE_SEQ_LEN, _HIDDEN))
    mask_ref[...] = (ids_ref[...] != _PAD_ID).astype(jnp.int8)


_mlp_mask = pl.pallas_call(
    _mlp_mask_body,
    out_shape=(
        jax.ShapeDtypeStruct((_BATCH, _PRE_SEQ_LEN, _HIDDEN), jnp.float32),
        jax.ShapeDtypeStruct((_BATCH, _SEQ), jnp.int8),
    ),
)


def kernel(input_ids, sentences_ids, W_embed, nomal_emb, W1, b1, W2, b2):
    ids32 = input_ids.astype(jnp.int32)
    flat = _gather_sc(ids32, W_embed)
    inputs_embeds = flat.reshape(_BATCH, _SEQ, _HIDDEN)
    prompts, mask8 = _mlp_mask(
        ids32, nomal_emb, W1, b1.reshape(1, _HIDDEN), W2, b2.reshape(1, _HIDDEN)
    )
    return inputs_embeds, prompts, mask8.astype(jnp.bool_)
